# baseline (speedup 1.0000x reference)
import functools
import math

import jax
import jax.numpy as jnp
from jax import lax
from jax.experimental import pallas as pl
from jax.experimental.pallas import tpu as pltpu
from jax.experimental.pallas import tpu_sc as plsc

ALPHA = 0.25
SMOOTH = 1e-05

B = 2
C = 4
DHW = 128 * 128 * 128
N = B * DHW
NC, NS = 2, 16
NW = NC * NS
K = 8192
LANES = 16

SC_CHUNKS = 6
W_TC = 65536

C1 = 1.0 - SMOOTH - SMOOTH / 3.0
C2 = SMOOTH / 3.0
LN2 = math.log(2.0)


def _focal_partials_sc(logit_flat, target_flat):
    perw = SC_CHUNKS * K
    mesh = plsc.VectorSubcoreMesh(core_axis_name="c", subcore_axis_name="s",
                                  num_cores=NC, num_subcores=NS)

    @functools.partial(
        pl.kernel,
        out_type=jax.ShapeDtypeStruct((NW * LANES,), jnp.float32),
        mesh=mesh,
        compiler_params=pltpu.CompilerParams(needs_layout_passes=False),
        scratch_types=[
            pltpu.VMEM((C * K,), jnp.float32),
            pltpu.VMEM((C * K,), jnp.float32),
            pltpu.VMEM((K,), jnp.int32),
            pltpu.VMEM((K,), jnp.int32),
            pltpu.VMEM((LANES,), jnp.float32),
            pltpu.SemaphoreType.DMA,
            pltpu.SemaphoreType.DMA,
        ],
    )
    def k(l_hbm, t_hbm, out_hbm, lb0, lb1, tb0, tb1, accb, sem0, sem1):
        wid = lax.axis_index("c") * NS + lax.axis_index("s")
        n0w = wid * perw
        lbufs = (lb0, lb1)
        tbufs = (tb0, tb1)
        sems = (sem0, sem1)

        def start(g):
            bank = g % 2
            n0 = n0w + g * K
            b = n0 >> 21
            p = n0 & (DHW - 1)
            cps = []
            for ch in range(C):
                off = pl.multiple_of((b * C + ch) * DHW + p, K)
                cps.append(pltpu.async_copy(
                    l_hbm.at[pl.ds(off, K)],
                    lbufs[bank].at[pl.ds(ch * K, K)],
                    sems[bank]))
            cps.append(pltpu.async_copy(
                t_hbm.at[pl.ds(pl.multiple_of(b * DHW + p, K), K)],
                tbufs[bank], sems[bank]))
            return cps

        iota = lax.iota(jnp.int32, LANES)

        def make_body(lb, tb):
            def body(i, acc):
                base = i * LANES
                t = tb[pl.ds(base, LANES)]
                l0 = lb[pl.ds(base, LANES)]
                l1 = lb[pl.ds(K + base, LANES)]
                l2 = lb[pl.ds(2 * K + base, LANES)]
                l3 = lb[pl.ds(3 * K + base, LANES)]
                lt = plsc.load_gather(lb, [t * K + (base + iota)])
                s = (l0 + l1) + (l2 + l3)
                pt = C1 * lt + C2 * s + SMOOTH
                at = jnp.where(t == 0, ALPHA, 1.0 - ALPHA)
                bits = plsc.bitcast(pt, jnp.int32)
                ef = ((bits >> 23) - 127).astype(jnp.float32)
                m = plsc.bitcast((bits & 0x007FFFFF) | 0x3F800000, jnp.float32)
                z = (m - 1.0) / (m + 1.0)
                z2 = z * z
                logpt = ef * LN2 + z * (
                    2.0 + z2 * (2.0 / 3.0 + z2 * (2.0 / 5.0 + z2 * (2.0 / 7.0))))
                omp = 1.0 - pt
                return acc - at * (omp * omp) * logpt
            return body

        acc = jnp.zeros((LANES,), jnp.float32)
        pending = {0: start(0)}
        for g in range(SC_CHUNKS):
            if g + 1 < SC_CHUNKS:
                pending[g + 1] = start(g + 1)
            for cp in pending.pop(g):
                cp.wait()
            acc = lax.fori_loop(0, K // LANES,
                                make_body(lbufs[g % 2], tbufs[g % 2]), acc)
        accb[...] = acc
        pltpu.sync_copy(accb, out_hbm.at[pl.ds(wid * LANES, LANES)])

    return k(logit_flat, target_flat)


def _focal_partial_tc(logit_flat, target_flat, n_start):
    n_blocks = (N - n_start) // W_TC
    g0 = n_start // W_TC
    pb = DHW // W_TC
    l_r = logit_flat.reshape(B * C * pb, 8, W_TC // 8)
    t_r = target_flat.reshape(B * pb, 8, W_TC // 8)

    def im_l(ch):
        def im(j):
            g = g0 + j
            return ((g // pb) * (C * pb) + ch * pb + g % pb, 0, 0)
        return im

    def body(t_ref, l0_ref, l1_ref, l2_ref, l3_ref, out_ref):
        j = pl.program_id(0)
        t = t_ref[0]
        l0, l1, l2, l3 = l0_ref[0], l1_ref[0], l2_ref[0], l3_ref[0]
        lt = jnp.where(t == 0, l0,
                       jnp.where(t == 1, l1, jnp.where(t == 2, l2, l3)))
        s = (l0 + l1) + (l2 + l3)
        pt = C1 * lt + C2 * s + SMOOTH
        at = jnp.where(t == 0, ALPHA, 1.0 - ALPHA)
        part = jnp.sum(at * jnp.square(1.0 - pt) * jnp.log(pt))

        @pl.when(j == 0)
        def _():
            out_ref[0, 0] = 0.0

        out_ref[0, 0] -= part

    blk = pl.BlockSpec((1, 8, W_TC // 8), lambda j: (0, 0, 0))
    return pl.pallas_call(
        body,
        grid=(n_blocks,),
        in_specs=[
            pl.BlockSpec((1, 8, W_TC // 8), lambda j: (g0 + j, 0, 0)),
            blk.replace(index_map=im_l(0)),
            blk.replace(index_map=im_l(1)),
            blk.replace(index_map=im_l(2)),
            blk.replace(index_map=im_l(3)),
        ],
        out_specs=pl.BlockSpec((1, 1), lambda j: (0, 0),
                               memory_space=pltpu.SMEM),
        out_shape=jax.ShapeDtypeStruct((1, 1), jnp.float32),
    )(t_r, l_r, l_r, l_r, l_r)


def kernel(logit, target):
    lf = logit.reshape(-1)
    tf = target.reshape(-1)
    n_sc = NW * SC_CHUNKS * K
    total = jnp.float32(0.0)
    if SC_CHUNKS > 0:
        total = total + jnp.sum(_focal_partials_sc(lf, tf))
    if n_sc < N:
        total = total + _focal_partial_tc(lf, tf, n_sc)[0, 0]
    return total / N


# speedup vs baseline: 1.6477x; 1.6477x over previous
import functools
import math

import jax
import jax.numpy as jnp
from jax import lax
from jax.experimental import pallas as pl
from jax.experimental.pallas import tpu as pltpu
from jax.experimental.pallas import tpu_sc as plsc

ALPHA = 0.25
SMOOTH = 1e-05

B = 2
C = 4
DHW = 128 * 128 * 128
N = B * DHW
NC, NS = 2, 16
NW = NC * NS
PERW = N // NW
K = 8192
NCHUNK = PERW // K
LANES = 16

C1 = 1.0 - SMOOTH - SMOOTH / 3.0
C2 = SMOOTH / 3.0
LN2 = math.log(2.0)
P0 = 0.40545697196626823
P1 = 0.33333566914933793
P2 = -0.05540978891397988
P3 = 0.012303837607177872
P4 = -0.003464705284351843
P5 = 0.0009315239138598057


def _focal_partials(logit_flat, target_flat):
    mesh = plsc.VectorSubcoreMesh(core_axis_name="c", subcore_axis_name="s",
                                  num_cores=NC, num_subcores=NS)

    @functools.partial(
        pl.kernel,
        out_type=jax.ShapeDtypeStruct((NW * LANES,), jnp.float32),
        mesh=mesh,
        compiler_params=pltpu.CompilerParams(needs_layout_passes=False),
        scratch_types=[
            pltpu.VMEM((C * K,), jnp.float32),
            pltpu.VMEM((C * K,), jnp.float32),
            pltpu.VMEM((K,), jnp.int32),
            pltpu.VMEM((K,), jnp.int32),
            pltpu.VMEM((LANES,), jnp.float32),
            pltpu.SemaphoreType.DMA,
            pltpu.SemaphoreType.DMA,
        ],
    )
    def k(l_hbm, t_hbm, out_hbm, lb0, lb1, tb0, tb1, accb, sem0, sem1):
        wid = lax.axis_index("c") * NS + lax.axis_index("s")
        b = wid // NS
        p0 = (wid % NS) * PERW
        lbufs = (lb0, lb1)
        tbufs = (tb0, tb1)
        sems = (sem0, sem1)

        def start(g):
            bank = g % 2
            cps = []
            for ch in range(C):
                off = (b * C + ch) * DHW + p0 + g * K
                cps.append(pltpu.async_copy(
                    l_hbm.at[pl.ds(off, K)],
                    lbufs[bank].at[pl.ds(ch * K, K)],
                    sems[bank]))
            cps.append(pltpu.async_copy(
                t_hbm.at[pl.ds(b * DHW + p0 + g * K, K)],
                tbufs[bank], sems[bank]))
            return cps

        iota = lax.iota(jnp.int32, LANES)

        def make_body(lb, tb):
            def body(i, acc):
                base = i * LANES
                t = tb[pl.ds(base, LANES)]
                l0 = lb[pl.ds(base, LANES)]
                l1 = lb[pl.ds(K + base, LANES)]
                l2 = lb[pl.ds(2 * K + base, LANES)]
                l3 = lb[pl.ds(3 * K + base, LANES)]
                lt = plsc.load_gather(lb, [t * K + (base + iota)])
                s = (l0 + l1) + (l2 + l3)
                pt = C1 * lt + (C2 * s + SMOOTH)
                at = jnp.where(t == 0, ALPHA, 1.0 - ALPHA)
                bits = plsc.bitcast(pt, jnp.int32)
                ef = ((bits >> 23) - 127).astype(jnp.float32)
                mant = bits & 0x007FFFFF
                y = plsc.bitcast(mant | 0x40000000, jnp.float32) - 3.0
                poly = P0 + y * (P1 + y * (P2 + y * (P3 + y * (P4 + y * P5))))
                logpt = ef * LN2 + poly
                omp = 1.0 - pt
                return acc - at * (omp * omp) * logpt
            return body

        acc = jnp.zeros((LANES,), jnp.float32)
        pending = {0: start(0)}
        for g in range(NCHUNK):
            if g + 1 < NCHUNK:
                pending[g + 1] = start(g + 1)
            for cp in pending.pop(g):
                cp.wait()
            acc = lax.fori_loop(0, K // LANES,
                                make_body(lbufs[g % 2], tbufs[g % 2]), acc)
        accb[...] = acc
        pltpu.sync_copy(accb, out_hbm.at[pl.ds(wid * LANES, LANES)])

    return k(logit_flat, target_flat)


def kernel(logit, target):
    partials = _focal_partials(logit.reshape(-1), target.reshape(-1))
    return jnp.sum(partials) / N


# speedup vs baseline: 1.8265x; 1.1085x over previous
import functools
import math

import jax
import jax.numpy as jnp
from jax import lax
from jax.experimental import pallas as pl
from jax.experimental.pallas import tpu as pltpu
from jax.experimental.pallas import tpu_sc as plsc

ALPHA = 0.25
SMOOTH = 1e-05

B = 2
C = 4
DHW = 128 * 128 * 128
N = B * DHW
NC, NS = 2, 16
NW = NC * NS
PERW = N // NW
K = 8192
NCHUNK = PERW // K
LANES = 16

C1 = 1.0 - SMOOTH - SMOOTH / 3.0
C2 = SMOOTH / 3.0
LN2 = math.log(2.0)
LOGA = LN2 / (1 << 23)
P0 = -87.97080041328486
P1 = -0.013525385405272464
P2 = -0.05547594402555503
P3 = 0.013463355084134012
P4 = -0.0033981833472911597
UNROLL = 4


def _focal_partials(logit_flat, target_flat):
    mesh = plsc.VectorSubcoreMesh(core_axis_name="c", subcore_axis_name="s",
                                  num_cores=NC, num_subcores=NS)

    @functools.partial(
        pl.kernel,
        out_type=jax.ShapeDtypeStruct((NW * LANES,), jnp.float32),
        mesh=mesh,
        compiler_params=pltpu.CompilerParams(needs_layout_passes=False),
        scratch_types=[
            pltpu.VMEM((C * K,), jnp.float32),
            pltpu.VMEM((C * K,), jnp.float32),
            pltpu.VMEM((K,), jnp.int32),
            pltpu.VMEM((K,), jnp.int32),
            pltpu.VMEM((LANES,), jnp.float32),
            pltpu.VMEM((LANES,), jnp.float32),
            pltpu.SemaphoreType.DMA,
            pltpu.SemaphoreType.DMA,
        ],
    )
    def k(l_hbm, t_hbm, out_hbm, lb0, lb1, tb0, tb1, accb, atb, sem0, sem1):
        wid = lax.axis_index("c") * NS + lax.axis_index("s")
        b = wid // NS
        p0 = (wid % NS) * PERW
        lbufs = (lb0, lb1)
        tbufs = (tb0, tb1)
        sems = (sem0, sem1)

        def start(g):
            bank = g % 2
            cps = []
            for ch in range(C):
                off = (b * C + ch) * DHW + p0 + g * K
                cps.append(pltpu.async_copy(
                    l_hbm.at[pl.ds(off, K)],
                    lbufs[bank].at[pl.ds(ch * K, K)],
                    sems[bank]))
            cps.append(pltpu.async_copy(
                t_hbm.at[pl.ds(b * DHW + p0 + g * K, K)],
                tbufs[bank], sems[bank]))
            return cps

        iota = lax.iota(jnp.int32, LANES)
        atb[...] = jnp.where(iota == 0, ALPHA, 1.0 - ALPHA)

        def one_vec(lb, tb, base):
            t = tb[pl.ds(base, LANES)]
            l0 = lb[pl.ds(base, LANES)]
            l1 = lb[pl.ds(K + base, LANES)]
            l2 = lb[pl.ds(2 * K + base, LANES)]
            l3 = lb[pl.ds(3 * K + base, LANES)]
            lt = plsc.load_gather(lb, [(t << 13) + (base + iota)])
            at = plsc.load_gather(atb, [t])
            s = (l0 + l1) + (l2 + l3)
            pt = C1 * lt + (C2 * s + SMOOTH)
            bits = plsc.bitcast(pt, jnp.int32)
            f = bits.astype(jnp.float32) * LOGA
            y = plsc.bitcast((bits & 0x007FFFFF) | 0x40000000, jnp.float32) - 3.0
            logpt = f + (P0 + y * (P1 + y * (P2 + y * (P3 + y * P4))))
            omp = 1.0 - pt
            return at * (omp * omp) * logpt

        def make_body(lb, tb):
            def body(i, accs):
                base = i * (LANES * UNROLL)
                return tuple(
                    accs[u] - one_vec(lb, tb, base + u * LANES)
                    for u in range(UNROLL))
            return body

        accs = (jnp.zeros((LANES,), jnp.float32),) * UNROLL
        pending = {0: start(0)}
        for g in range(NCHUNK):
            if g + 1 < NCHUNK:
                pending[g + 1] = start(g + 1)
            for cp in pending.pop(g):
                cp.wait()
            accs = lax.fori_loop(0, K // (LANES * UNROLL),
                                 make_body(lbufs[g % 2], tbufs[g % 2]), accs)
        accb[...] = (accs[0] + accs[1]) + (accs[2] + accs[3])
        pltpu.sync_copy(accb, out_hbm.at[pl.ds(wid * LANES, LANES)])

    return k(logit_flat, target_flat)


def kernel(logit, target):
    partials = _focal_partials(logit.reshape(-1), target.reshape(-1))
    return jnp.sum(partials) / N
